# baseline (speedup 1.0000x reference)
import functools

import jax
import jax.numpy as jnp
from jax import lax
from jax.experimental import pallas as pl
from jax.experimental.pallas import tpu as pltpu
from jax.experimental.pallas import tpu_sc as plsc

N_NODES = 10000
LAT = 32
N_ITERS = 3
NC = 2
NS = 16
NW = NC * NS
B = 125
ROWS = 80
PAD_NODES = N_NODES
STRIPE = PAD_NODES // NS
DEG_PAD = 10240
DEG_STRIPE = DEG_PAD // NS


def _vmesh():
    return plsc.VectorSubcoreMesh(core_axis_name="c", subcore_axis_name="s")


_SC_PARAMS = pltpu.CompilerParams(use_tc_tiling_on_sc=False)


def _sc_agg(proc, src3, dst3, zeros_stripe):
    @functools.partial(
        pl.kernel,
        out_type=jax.ShapeDtypeStruct((NC, PAD_NODES, LAT), jnp.float32),
        mesh=_vmesh(),
        scratch_types=[
            pltpu.VMEM((ROWS, B), jnp.int32),
            pltpu.VMEM((ROWS, B), jnp.int32),
            pltpu.VMEM((B, LAT), jnp.float32),
            pltpu.VMEM((B, LAT), jnp.float32),
            pltpu.VMEM((B, LAT), jnp.float32),
            pltpu.VMEM((B, LAT), jnp.float32),
            pltpu.VMEM_SHARED((PAD_NODES, LAT), jnp.float32),
            pltpu.SemaphoreType.DMA,
            pltpu.SemaphoreType.DMA,
            pltpu.SemaphoreType.DMA,
            pltpu.SemaphoreType.DMA,
        ],
        compiler_params=_SC_PARAMS,
    )
    def k(proc_hbm, src_hbm, dst_hbm, z_hbm, out_hbm,
          srcv, dstv, rows0, rows1, rows2, rows3, acc, sem0, sem1, sem2, sem3):
        c = lax.axis_index("c")
        s = lax.axis_index("s")
        w = c * NS + s
        pltpu.sync_copy(z_hbm, acc.at[pl.ds(s * STRIPE, STRIPE)])
        pltpu.sync_copy(src_hbm.at[w], srcv)
        pltpu.sync_copy(dst_hbm.at[w], dstv)
        plsc.subcore_barrier()

        bufs = [(rows0, sem0), (rows1, sem1), (rows2, sem2), (rows3, sem3)]
        nb = len(bufs)

        def start(j, b):
            pltpu.async_copy(proc_hbm.at[srcv.at[j]], bufs[b][0], bufs[b][1])

        def drain(j, b):
            pltpu.make_async_copy(proc_hbm.at[srcv.at[j]],
                                  bufs[b][0], bufs[b][1]).wait()
            pltpu.sync_copy(bufs[b][0], acc.at[dstv.at[j]], add=True)

        for b in range(nb - 1):
            start(b, b)

        def body(jj, carry):
            j = nb * jj
            for b in range(nb):
                start(j + b + nb - 1, (b + nb - 1) % nb)
                drain(j + b, b)
            return carry

        lax.fori_loop(0, ROWS // nb - 1, body, 0)
        j = ROWS - nb
        start(ROWS - 1, nb - 1)
        for b in range(nb):
            drain(j + b, b)
        plsc.subcore_barrier()
        pltpu.sync_copy(acc.at[pl.ds(s * STRIPE, STRIPE)],
                        out_hbm.at[c].at[pl.ds(s * STRIPE, STRIPE)])

    return k(proc, src3, dst3, zeros_stripe)


def _sc_deg(dst3, zeros_deg, ones_row):
    @functools.partial(
        pl.kernel,
        out_type=jax.ShapeDtypeStruct((NC, DEG_PAD), jnp.float32),
        mesh=_vmesh(),
        scratch_types=[
            pltpu.VMEM((ROWS, B), jnp.int32),
            pltpu.VMEM((B,), jnp.float32),
            pltpu.VMEM_SHARED((DEG_PAD,), jnp.float32),
        ],
        compiler_params=_SC_PARAMS,
    )
    def k(dst_hbm, z_hbm, ones_hbm, out_hbm, dstv, onesv, acc):
        c = lax.axis_index("c")
        s = lax.axis_index("s")
        w = c * NS + s
        pltpu.sync_copy(z_hbm, acc.at[pl.ds(s * DEG_STRIPE, DEG_STRIPE)])
        pltpu.sync_copy(dst_hbm.at[w], dstv)
        pltpu.sync_copy(ones_hbm, onesv)
        plsc.subcore_barrier()

        def body(j, carry):
            pltpu.sync_copy(onesv, acc.at[dstv.at[j]], add=True)
            return carry

        lax.fori_loop(0, ROWS, body, 0)
        plsc.subcore_barrier()
        pltpu.sync_copy(acc.at[pl.ds(s * DEG_STRIPE, DEG_STRIPE)],
                        out_hbm.at[c].at[pl.ds(s * DEG_STRIPE, DEG_STRIPE)])

    return k(dst3, zeros_deg, ones_row)


def _lrelu(x):
    return jnp.where(x >= 0, x, 0.01 * x)


def _row2(v):
    return v.reshape(1, -1)


def _enc_call(x, p):
    def body(x_ref, w0, b0, w1, b1, w2, b2, g, bt, o_ref):
        h = _lrelu(x_ref[...] @ w0[...] + b0[...])
        h = _lrelu(h @ w1[...] + b1[...])
        h = h @ w2[...] + b2[...]
        mu = jnp.mean(h, axis=-1, keepdims=True)
        var = jnp.mean((h - mu) ** 2, axis=-1, keepdims=True)
        o_ref[...] = (h - mu) * lax.rsqrt(var + 1e-5) * g[...] + bt[...]

    return pl.pallas_call(
        body,
        out_shape=jax.ShapeDtypeStruct((N_NODES, LAT), jnp.float32),
    )(x, p['Win'], _row2(p['bin']), p['Wh'][0], _row2(p['bh'][0]),
      p['Wout'], _row2(p['bout']), _row2(p['gamma']), _row2(p['beta']))


def _proc_call(proc, S, deg, p):
    def body(x_ref, s_ref, d_ref, wa, wb, b0, w1, b1, w2, b2, g, bt, o_ref):
        x = x_ref[...]
        pe = (s_ref[0] + s_ref[1]
              + (d_ref[0, :N_NODES] + d_ref[1, :N_NODES]) * x)
        h = _lrelu(x @ wa[...] + pe @ wb[...] + b0[...])
        h = _lrelu(h @ w1[...] + b1[...])
        h = h @ w2[...] + b2[...]
        mu = jnp.mean(h, axis=-1, keepdims=True)
        var = jnp.mean((h - mu) ** 2, axis=-1, keepdims=True)
        o_ref[...] = (h - mu) * lax.rsqrt(var + 1e-5) * g[...] + bt[...] + x

    return pl.pallas_call(
        body,
        out_shape=jax.ShapeDtypeStruct((N_NODES, LAT), jnp.float32),
    )(proc, S, deg, p['Win'][:LAT], p['Win'][LAT:], _row2(p['bin']),
      p['Wh'][0], _row2(p['bh'][0]), p['Wout'], _row2(p['bout']),
      _row2(p['gamma']), _row2(p['beta']))


def _out_call(proc, p):
    def body(x_ref, w0, b0, w1, b1, w2, b2, o_ref):
        h = _lrelu(x_ref[...] @ w0[...] + b0[...])
        h = _lrelu(h @ w1[...] + b1[...])
        o_ref[...] = h @ w2[...] + b2[...]

    return pl.pallas_call(
        body,
        out_shape=jax.ShapeDtypeStruct((N_NODES, p['Wout'].shape[1]), jnp.float32),
    )(proc, p['Win'], _row2(p['bin']), p['Wh'][0], _row2(p['bh'][0]),
      p['Wout'], _row2(p['bout']))


def kernel(in_feat, edge_index, params):
    src = edge_index[0]
    dst = edge_index[1]
    src3 = src.reshape(NW, ROWS, B)
    dst3 = dst.reshape(NW, ROWS, B)
    zeros_s = jnp.zeros((STRIPE, LAT), jnp.float32)
    zeros_d = jnp.zeros((DEG_STRIPE,), jnp.float32)
    ones_r = jnp.ones((B,), jnp.float32)

    deg = _sc_deg(dst3, zeros_d, ones_r).reshape(NC, DEG_PAD, 1)
    proc = _enc_call(in_feat, params['enc'])
    for i in range(N_ITERS):
        S = _sc_agg(proc, src3, dst3, zeros_s)
        proc = _proc_call(proc, S, deg, params['proc'][i])
    return _out_call(proc, params['out'])


# speedup vs baseline: 1.0337x; 1.0337x over previous
import functools

import jax
import jax.numpy as jnp
from jax import lax
from jax.experimental import pallas as pl
from jax.experimental.pallas import tpu as pltpu
from jax.experimental.pallas import tpu_sc as plsc

N_NODES = 10000
LAT = 32
N_ITERS = 3
NC = 2
NS = 16
NW = NC * NS
B = 125
ROWS = 80
PAD_NODES = N_NODES
STRIPE = PAD_NODES // NS
DEG_PAD = 10240
DEG_STRIPE = DEG_PAD // NS


def _vmesh():
    return plsc.VectorSubcoreMesh(core_axis_name="c", subcore_axis_name="s")


_SC_PARAMS = pltpu.CompilerParams(use_tc_tiling_on_sc=False)


def _sc_agg(proc, src3, dst3, zeros_stripe):
    @functools.partial(
        pl.kernel,
        out_type=jax.ShapeDtypeStruct((NC, PAD_NODES, LAT), jnp.float32),
        mesh=_vmesh(),
        scratch_types=[
            pltpu.VMEM((ROWS, B), jnp.int32),
            pltpu.VMEM((ROWS, B), jnp.int32),
            pltpu.VMEM((B, LAT), jnp.float32),
            pltpu.VMEM((B, LAT), jnp.float32),
            pltpu.VMEM((B, LAT), jnp.float32),
            pltpu.VMEM((B, LAT), jnp.float32),
            pltpu.VMEM((B, LAT), jnp.float32),
            pltpu.VMEM((B, LAT), jnp.float32),
            pltpu.VMEM((B, LAT), jnp.float32),
            pltpu.VMEM((B, LAT), jnp.float32),
            pltpu.VMEM_SHARED((PAD_NODES, LAT), jnp.float32),
            pltpu.SemaphoreType.DMA,
            pltpu.SemaphoreType.DMA,
            pltpu.SemaphoreType.DMA,
            pltpu.SemaphoreType.DMA,
            pltpu.SemaphoreType.DMA,
            pltpu.SemaphoreType.DMA,
            pltpu.SemaphoreType.DMA,
            pltpu.SemaphoreType.DMA,
        ],
        compiler_params=_SC_PARAMS,
    )
    def k(proc_hbm, src_hbm, dst_hbm, z_hbm, out_hbm,
          srcv, dstv, rows0, rows1, rows2, rows3, rows4, rows5, rows6, rows7,
          acc, sem0, sem1, sem2, sem3, sem4, sem5, sem6, sem7):
        c = lax.axis_index("c")
        s = lax.axis_index("s")
        w = c * NS + s
        pltpu.sync_copy(z_hbm, acc.at[pl.ds(s * STRIPE, STRIPE)])
        pltpu.sync_copy(src_hbm.at[w], srcv)
        pltpu.sync_copy(dst_hbm.at[w], dstv)
        plsc.subcore_barrier()

        bufs = [(rows0, sem0), (rows1, sem1), (rows2, sem2), (rows3, sem3),
                (rows4, sem4), (rows5, sem5), (rows6, sem6), (rows7, sem7)]
        nb = len(bufs)

        def start(j, b):
            pltpu.async_copy(proc_hbm.at[srcv.at[j]], bufs[b][0], bufs[b][1])

        def drain(j, b):
            pltpu.make_async_copy(proc_hbm.at[srcv.at[j]],
                                  bufs[b][0], bufs[b][1]).wait()
            pltpu.sync_copy(bufs[b][0], acc.at[dstv.at[j]], add=True)

        for b in range(nb - 1):
            start(b, b)

        def body(jj, carry):
            j = nb * jj
            for b in range(nb):
                start(j + b + nb - 1, (b + nb - 1) % nb)
                drain(j + b, b)
            return carry

        lax.fori_loop(0, ROWS // nb - 1, body, 0)
        j = ROWS - nb
        start(ROWS - 1, nb - 1)
        for b in range(nb):
            drain(j + b, b)
        plsc.subcore_barrier()
        pltpu.sync_copy(acc.at[pl.ds(s * STRIPE, STRIPE)],
                        out_hbm.at[c].at[pl.ds(s * STRIPE, STRIPE)])

    return k(proc, src3, dst3, zeros_stripe)


def _sc_deg(dst3, zeros_deg, ones_row):
    @functools.partial(
        pl.kernel,
        out_type=jax.ShapeDtypeStruct((NC, DEG_PAD), jnp.float32),
        mesh=_vmesh(),
        scratch_types=[
            pltpu.VMEM((ROWS, B), jnp.int32),
            pltpu.VMEM((B,), jnp.float32),
            pltpu.VMEM_SHARED((DEG_PAD,), jnp.float32),
        ],
        compiler_params=_SC_PARAMS,
    )
    def k(dst_hbm, z_hbm, ones_hbm, out_hbm, dstv, onesv, acc):
        c = lax.axis_index("c")
        s = lax.axis_index("s")
        w = c * NS + s
        pltpu.sync_copy(z_hbm, acc.at[pl.ds(s * DEG_STRIPE, DEG_STRIPE)])
        pltpu.sync_copy(dst_hbm.at[w], dstv)
        pltpu.sync_copy(ones_hbm, onesv)
        plsc.subcore_barrier()

        def body(j, carry):
            pltpu.sync_copy(onesv, acc.at[dstv.at[j]], add=True)
            return carry

        lax.fori_loop(0, ROWS, body, 0)
        plsc.subcore_barrier()
        pltpu.sync_copy(acc.at[pl.ds(s * DEG_STRIPE, DEG_STRIPE)],
                        out_hbm.at[c].at[pl.ds(s * DEG_STRIPE, DEG_STRIPE)])

    return k(dst3, zeros_deg, ones_row)


def _lrelu(x):
    return jnp.where(x >= 0, x, 0.01 * x)


def _row2(v):
    return v.reshape(1, -1)


def _enc_call(x, p):
    def body(x_ref, w0, b0, w1, b1, w2, b2, g, bt, o_ref):
        h = _lrelu(x_ref[...] @ w0[...] + b0[...])
        h = _lrelu(h @ w1[...] + b1[...])
        h = h @ w2[...] + b2[...]
        mu = jnp.mean(h, axis=-1, keepdims=True)
        var = jnp.mean((h - mu) ** 2, axis=-1, keepdims=True)
        o_ref[...] = (h - mu) * lax.rsqrt(var + 1e-5) * g[...] + bt[...]

    return pl.pallas_call(
        body,
        out_shape=jax.ShapeDtypeStruct((N_NODES, LAT), jnp.float32),
    )(x, p['Win'], _row2(p['bin']), p['Wh'][0], _row2(p['bh'][0]),
      p['Wout'], _row2(p['bout']), _row2(p['gamma']), _row2(p['beta']))


def _proc_call(proc, S, deg, p):
    def body(x_ref, s_ref, d_ref, wa, wb, b0, w1, b1, w2, b2, g, bt, o_ref):
        x = x_ref[...]
        pe = (s_ref[0] + s_ref[1]
              + (d_ref[0, :N_NODES] + d_ref[1, :N_NODES]) * x)
        h = _lrelu(x @ wa[...] + pe @ wb[...] + b0[...])
        h = _lrelu(h @ w1[...] + b1[...])
        h = h @ w2[...] + b2[...]
        mu = jnp.mean(h, axis=-1, keepdims=True)
        var = jnp.mean((h - mu) ** 2, axis=-1, keepdims=True)
        o_ref[...] = (h - mu) * lax.rsqrt(var + 1e-5) * g[...] + bt[...] + x

    return pl.pallas_call(
        body,
        out_shape=jax.ShapeDtypeStruct((N_NODES, LAT), jnp.float32),
    )(proc, S, deg, p['Win'][:LAT], p['Win'][LAT:], _row2(p['bin']),
      p['Wh'][0], _row2(p['bh'][0]), p['Wout'], _row2(p['bout']),
      _row2(p['gamma']), _row2(p['beta']))


def _out_call(proc, p):
    def body(x_ref, w0, b0, w1, b1, w2, b2, o_ref):
        h = _lrelu(x_ref[...] @ w0[...] + b0[...])
        h = _lrelu(h @ w1[...] + b1[...])
        o_ref[...] = h @ w2[...] + b2[...]

    return pl.pallas_call(
        body,
        out_shape=jax.ShapeDtypeStruct((N_NODES, p['Wout'].shape[1]), jnp.float32),
    )(proc, p['Win'], _row2(p['bin']), p['Wh'][0], _row2(p['bh'][0]),
      p['Wout'], _row2(p['bout']))


def kernel(in_feat, edge_index, params):
    src = edge_index[0]
    dst = edge_index[1]
    src3 = src.reshape(NW, ROWS, B)
    dst3 = dst.reshape(NW, ROWS, B)
    zeros_s = jnp.zeros((STRIPE, LAT), jnp.float32)
    zeros_d = jnp.zeros((DEG_STRIPE,), jnp.float32)
    ones_r = jnp.ones((B,), jnp.float32)

    deg = _sc_deg(dst3, zeros_d, ones_r).reshape(NC, DEG_PAD, 1)
    proc = _enc_call(in_feat, params['enc'])
    for i in range(N_ITERS):
        S = _sc_agg(proc, src3, dst3, zeros_s)
        proc = _proc_call(proc, S, deg, params['proc'][i])
    return _out_call(proc, params['out'])


# speedup vs baseline: 1.2744x; 1.2328x over previous
import functools

import jax
import jax.numpy as jnp
from jax import lax
from jax.experimental import pallas as pl
from jax.experimental.pallas import tpu as pltpu
from jax.experimental.pallas import tpu_sc as plsc

N_NODES = 10000
LAT = 32
N_ITERS = 3
NC = 2
NS = 16
NW = NC * NS
B = 125
ROWS = 80
PAD_NODES = N_NODES
STRIPE = PAD_NODES // NS
DEG_PAD = 10240
DEG_STRIPE = DEG_PAD // NS


def _vmesh():
    return plsc.VectorSubcoreMesh(core_axis_name="c", subcore_axis_name="s")


_SC_PARAMS = pltpu.CompilerParams(use_tc_tiling_on_sc=False)


def _sc_agg(proc, src3, dst3, zeros_stripe):
    @functools.partial(
        pl.kernel,
        out_type=jax.ShapeDtypeStruct((NC, PAD_NODES, LAT), jnp.float32),
        mesh=_vmesh(),
        scratch_types=[
            pltpu.VMEM((ROWS, B), jnp.int32),
            pltpu.VMEM((ROWS, B), jnp.int32),
            pltpu.VMEM((B, LAT), jnp.float32),
            pltpu.VMEM((B, LAT), jnp.float32),
            pltpu.VMEM((B, LAT), jnp.float32),
            pltpu.VMEM((B, LAT), jnp.float32),
            pltpu.VMEM((B, LAT), jnp.float32),
            pltpu.VMEM((B, LAT), jnp.float32),
            pltpu.VMEM((B, LAT), jnp.float32),
            pltpu.VMEM((B, LAT), jnp.float32),
            pltpu.VMEM_SHARED((PAD_NODES, LAT), jnp.float32),
            pltpu.SemaphoreType.DMA,
            pltpu.SemaphoreType.DMA,
            pltpu.SemaphoreType.DMA,
            pltpu.SemaphoreType.DMA,
            pltpu.SemaphoreType.DMA,
            pltpu.SemaphoreType.DMA,
            pltpu.SemaphoreType.DMA,
            pltpu.SemaphoreType.DMA,
        ],
        compiler_params=_SC_PARAMS,
    )
    def k(proc_hbm, src_hbm, dst_hbm, z_hbm, out_hbm,
          srcv, dstv, rows0, rows1, rows2, rows3, rows4, rows5, rows6, rows7,
          acc, sem0, sem1, sem2, sem3, sem4, sem5, sem6, sem7):
        c = lax.axis_index("c")
        s = lax.axis_index("s")
        w = c * NS + s
        pltpu.sync_copy(z_hbm, acc.at[pl.ds(s * STRIPE, STRIPE)])
        pltpu.sync_copy(src_hbm.at[w], srcv)
        pltpu.sync_copy(dst_hbm.at[w], dstv)
        plsc.subcore_barrier()

        bufs = [(rows0, sem0), (rows1, sem1), (rows2, sem2), (rows3, sem3),
                (rows4, sem4), (rows5, sem5), (rows6, sem6), (rows7, sem7)]
        nb = len(bufs)

        def start(j, b):
            pltpu.async_copy(proc_hbm.at[srcv.at[j]], bufs[b][0], bufs[b][1])

        def drain(j, b):
            pltpu.make_async_copy(proc_hbm.at[srcv.at[j]],
                                  bufs[b][0], bufs[b][1]).wait()
            pltpu.sync_copy(bufs[b][0], acc.at[dstv.at[j]], add=True)

        for b in range(nb - 1):
            start(b, b)

        def body(jj, carry):
            j = nb * jj
            for b in range(nb):
                start(j + b + nb - 1, (b + nb - 1) % nb)
                drain(j + b, b)
            return carry

        lax.fori_loop(0, ROWS // nb - 1, body, 0)
        j = ROWS - nb
        start(ROWS - 1, nb - 1)
        for b in range(nb):
            drain(j + b, b)
        plsc.subcore_barrier()
        pltpu.sync_copy(acc.at[pl.ds(s * STRIPE, STRIPE)],
                        out_hbm.at[c].at[pl.ds(s * STRIPE, STRIPE)])

    return k(proc, src3, dst3, zeros_stripe)


def _sc_deg(dst3, zeros_deg, ones_row):
    @functools.partial(
        pl.kernel,
        out_type=jax.ShapeDtypeStruct((NC, DEG_PAD), jnp.float32),
        mesh=_vmesh(),
        scratch_types=[
            pltpu.VMEM((ROWS, B), jnp.int32),
            pltpu.VMEM((B,), jnp.float32),
            pltpu.VMEM_SHARED((DEG_PAD,), jnp.float32),
        ],
        compiler_params=_SC_PARAMS,
    )
    def k(dst_hbm, z_hbm, ones_hbm, out_hbm, dstv, onesv, acc):
        c = lax.axis_index("c")
        s = lax.axis_index("s")
        w = c * NS + s
        pltpu.sync_copy(z_hbm, acc.at[pl.ds(s * DEG_STRIPE, DEG_STRIPE)])
        pltpu.sync_copy(dst_hbm.at[w], dstv)
        pltpu.sync_copy(ones_hbm, onesv)
        plsc.subcore_barrier()

        def body(j, carry):
            pltpu.sync_copy(onesv, acc.at[dstv.at[j]], add=True)
            return carry

        lax.fori_loop(0, ROWS, body, 0)
        plsc.subcore_barrier()
        pltpu.sync_copy(acc.at[pl.ds(s * DEG_STRIPE, DEG_STRIPE)],
                        out_hbm.at[c].at[pl.ds(s * DEG_STRIPE, DEG_STRIPE)])

    return k(dst3, zeros_deg, ones_row)


PACK = 4
NP = N_NODES // PACK
PW = PACK * LAT


def _lrelu(x):
    return jnp.where(x >= 0, x, 0.01 * x)


def _row2(v):
    return v.reshape(1, -1)


def _kron4(w):
    return jnp.kron(jnp.eye(PACK, dtype=jnp.float32), w)


def _tile4(v):
    return jnp.tile(v, PACK).reshape(1, -1)


def _ln_mat():
    return _kron4(jnp.full((LAT, LAT), 1.0 / LAT, jnp.float32))


def _enc_call(x_p, p, m):
    def body(x_ref, w0, b0, w1, b1, w2, b2, g, bt, m_ref, o_ref):
        h = _lrelu(x_ref[...] @ w0[...] + b0[...])
        h = _lrelu(h @ w1[...] + b1[...])
        h = h @ w2[...] + b2[...]
        mu = h @ m_ref[...]
        cen = h - mu
        var = (cen * cen) @ m_ref[...]
        o_ref[...] = cen * lax.rsqrt(var + 1e-5) * g[...] + bt[...]

    return pl.pallas_call(
        body,
        out_shape=jax.ShapeDtypeStruct((NP, PW), jnp.float32),
    )(x_p, _kron4(p['Win']), _tile4(p['bin']), _kron4(p['Wh'][0]),
      _tile4(p['bh'][0]), _kron4(p['Wout']), _tile4(p['bout']),
      _tile4(p['gamma']), _tile4(p['beta']), m)


def _proc_call(proc_p, S_p, deg_p, p, m):
    def body(x_ref, s_ref, d_ref, wa, wb, b0, w1, b1, w2, b2, g, bt, m_ref,
             o_ref):
        x = x_ref[...]
        pe = s_ref[0] + s_ref[1] + d_ref[...] * x
        h = _lrelu(x @ wa[...] + pe @ wb[...] + b0[...])
        h = _lrelu(h @ w1[...] + b1[...])
        h = h @ w2[...] + b2[...]
        mu = h @ m_ref[...]
        cen = h - mu
        var = (cen * cen) @ m_ref[...]
        o_ref[...] = cen * lax.rsqrt(var + 1e-5) * g[...] + bt[...] + x

    return pl.pallas_call(
        body,
        out_shape=jax.ShapeDtypeStruct((NP, PW), jnp.float32),
    )(proc_p, S_p, deg_p, _kron4(p['Win'][:LAT]), _kron4(p['Win'][LAT:]),
      _tile4(p['bin']), _kron4(p['Wh'][0]), _tile4(p['bh'][0]),
      _kron4(p['Wout']), _tile4(p['bout']), _tile4(p['gamma']),
      _tile4(p['beta']), m)


def _out_call(proc_p, p):
    fout = p['Wout'].shape[1]

    def body(x_ref, w0, b0, w1, b1, w2, b2, o_ref):
        h = _lrelu(x_ref[...] @ w0[...] + b0[...])
        h = _lrelu(h @ w1[...] + b1[...])
        o_ref[...] = h @ w2[...] + b2[...]

    return pl.pallas_call(
        body,
        out_shape=jax.ShapeDtypeStruct((NP, PACK * fout), jnp.float32),
    )(proc_p, _kron4(p['Win']), _tile4(p['bin']), _kron4(p['Wh'][0]),
      _tile4(p['bh'][0]), _kron4(p['Wout']), _tile4(p['bout']))


def kernel(in_feat, edge_index, params):
    src = edge_index[0]
    dst = edge_index[1]
    src3 = src.reshape(NW, ROWS, B)
    dst3 = dst.reshape(NW, ROWS, B)
    zeros_s = jnp.zeros((STRIPE, LAT), jnp.float32)
    zeros_d = jnp.zeros((DEG_STRIPE,), jnp.float32)
    ones_r = jnp.ones((B,), jnp.float32)
    m = _ln_mat()

    deg = _sc_deg(dst3, zeros_d, ones_r)
    dsum = deg[0, :N_NODES] + deg[1, :N_NODES]
    deg_p = jnp.broadcast_to(dsum[:, None], (N_NODES, LAT)).reshape(NP, PW)

    fin = in_feat.shape[1]
    proc_p = _enc_call(in_feat.reshape(NP, PACK * fin), params['enc'], m)
    for i in range(N_ITERS):
        S = _sc_agg(proc_p.reshape(N_NODES, LAT), src3, dst3, zeros_s)
        proc_p = _proc_call(proc_p, S.reshape(NC, NP, PW), deg_p,
                            params['proc'][i], m)
    out_p = _out_call(proc_p, params['out'])
    return out_p.reshape(N_NODES, params['out']['Wout'].shape[1])
